# baseline (speedup 1.0000x reference)
import functools

import jax
import jax.numpy as jnp
from jax.experimental import pallas as pl
from jax.experimental.pallas import tpu as pltpu

LANE = 128
NUM_OUT = 64
VAL_LANE = 64
MAX_BATCH_TILE = 1024


def _round_up(x, m):
    return ((x + m - 1) // m) * m


def _choose_tile(B):
    B8 = _round_up(max(B, 1), 8)
    tb = min(MAX_BATCH_TILE, B8)
    if B8 // tb < 2 and B8 >= 16:
        tb = _round_up((B8 + 1) // 2, 8)
    B_pad = _round_up(B8, tb)
    return tb, B_pad


def _fused_kernel(obs_ref, cobs_ref, w1p_ref, w1v_ref, b1_ref,
                  w2_ref, b2_ref, w3_ref, b3_ref, logits_ref, vals_ref):
    obs = obs_ref[...].astype(jnp.bfloat16)
    cobs = cobs_ref[...].astype(jnp.bfloat16)
    pre = (jnp.dot(obs, w1p_ref[...], preferred_element_type=jnp.float32)
           + jnp.dot(cobs, w1v_ref[...], preferred_element_type=jnp.float32))
    h1 = jnp.tanh(pre + b1_ref[...]).astype(jnp.bfloat16)
    h2 = jnp.tanh(
        jnp.dot(h1, w2_ref[...], preferred_element_type=jnp.float32)
        + b2_ref[...]).astype(jnp.bfloat16)
    out = jnp.dot(h2, w3_ref[...], preferred_element_type=jnp.float32) + b3_ref[...]
    logits_ref[...] = out[:, :NUM_OUT]
    vals_ref[...] = out[:, VAL_LANE:VAL_LANE + 1]


@jax.jit
def _impl(obs, cc_obs, W1p, W1v, B1, W2, B2, W3, B3):
    obs = obs.astype(jnp.float32).reshape(obs.shape[0], -1)
    cobs = cc_obs.astype(jnp.float32).reshape(cc_obs.shape[0], -1)
    B = obs.shape[0]
    tb, B_pad = _choose_tile(B)
    if B_pad != B:
        obs = jnp.pad(obs, ((0, B_pad - B), (0, 0)))
        cobs = jnp.pad(cobs, ((0, B_pad - B), (0, 0)))

    w1p = W1p.astype(jnp.bfloat16)
    w1v = W1v.astype(jnp.bfloat16)
    w2 = W2.astype(jnp.bfloat16)
    w3 = W3.astype(jnp.bfloat16)

    logits, vals = pl.pallas_call(
        _fused_kernel,
        out_shape=(
            jax.ShapeDtypeStruct((B_pad, NUM_OUT), jnp.float32),
            jax.ShapeDtypeStruct((B_pad, 1), jnp.float32),
        ),
        grid=(B_pad // tb,),
        in_specs=[
            pl.BlockSpec((tb, obs.shape[1]), lambda i: (i, 0)),
            pl.BlockSpec((tb, cobs.shape[1]), lambda i: (i, 0)),
            pl.BlockSpec(w1p.shape, lambda i: (0, 0)),
            pl.BlockSpec(w1v.shape, lambda i: (0, 0)),
            pl.BlockSpec((1, LANE), lambda i: (0, 0)),
            pl.BlockSpec((LANE, LANE), lambda i: (0, 0)),
            pl.BlockSpec((1, LANE), lambda i: (0, 0)),
            pl.BlockSpec((LANE, LANE), lambda i: (0, 0)),
            pl.BlockSpec((1, LANE), lambda i: (0, 0)),
        ],
        out_specs=(
            pl.BlockSpec((tb, NUM_OUT), lambda i: (i, 0)),
            pl.BlockSpec((tb, 1), lambda i: (i, 0)),
        ),
        compiler_params=pltpu.CompilerParams(
            dimension_semantics=("parallel",),
        ),
    )(obs, cobs, w1p, w1v, B1, w2, B2, w3, B3)

    return logits[:B], vals[:B].reshape(-1)


def kernel(obs, cc_obs, W1p, W1v, B1, W2, B2, W3, B3):
    return _impl(obs, cc_obs, W1p, W1v, B1, W2, B2, W3, B3)


# speedup vs baseline: 1.3102x; 1.3102x over previous
import functools

import jax
import jax.numpy as jnp
from jax.experimental import pallas as pl
from jax.experimental.pallas import tpu as pltpu

LANE = 128
NUM_OUT = 64
VAL_LANE = 64
MAX_BATCH_TILE = 1024


def _round_up(x, m):
    return ((x + m - 1) // m) * m


def _choose_tile(B):
    B128 = _round_up(max(B, 1), LANE)
    tb = min(MAX_BATCH_TILE, B128)
    if B128 // tb < 2 and B128 >= 2 * LANE:
        tb = _round_up((B128 + 1) // 2, LANE)
    B_pad = _round_up(B128, tb)
    return tb, B_pad


def _fused_kernel(obs_ref, cobs_ref, w1p_ref, w1v_ref, b1_ref,
                  w2_ref, b2_ref, w3_ref, b3_ref, logits_ref, vals_ref):
    tb = obs_ref.shape[0]
    obs = obs_ref[...].astype(jnp.bfloat16)
    cobs = cobs_ref[...].astype(jnp.bfloat16)
    w1p = w1p_ref[...].astype(jnp.bfloat16)
    w1v = w1v_ref[...].astype(jnp.bfloat16)
    w2 = w2_ref[...].astype(jnp.bfloat16)
    w3 = w3_ref[...].astype(jnp.bfloat16)
    pre = (jnp.dot(obs, w1p, preferred_element_type=jnp.float32)
           + jnp.dot(cobs, w1v, preferred_element_type=jnp.float32))
    h1 = jnp.tanh(pre + b1_ref[...]).astype(jnp.bfloat16)
    h2 = jnp.tanh(
        jnp.dot(h1, w2, preferred_element_type=jnp.float32)
        + b2_ref[...]).astype(jnp.bfloat16)
    logits_ref[...] = (
        jnp.dot(h2, w3[:, :NUM_OUT], preferred_element_type=jnp.float32)
        + b3_ref[0, :NUM_OUT])
    w3v_row = w3[:, VAL_LANE:VAL_LANE + 1].reshape(1, LANE)
    v_row = jax.lax.dot_general(
        w3v_row, h2, dimension_numbers=(((1,), (1,)), ((), ())),
        preferred_element_type=jnp.float32)
    vals_ref[...] = v_row.reshape(tb // LANE, LANE) + b3_ref[0, VAL_LANE]


@jax.jit
def _impl(obs, cc_obs, W1p, W1v, B1, W2, B2, W3, B3):
    obs = obs.astype(jnp.float32).reshape(obs.shape[0], -1)
    cobs = cc_obs.astype(jnp.float32).reshape(cc_obs.shape[0], -1)
    B = obs.shape[0]
    tb, B_pad = _choose_tile(B)
    if B_pad != B:
        obs = jnp.pad(obs, ((0, B_pad - B), (0, 0)))
        cobs = jnp.pad(cobs, ((0, B_pad - B), (0, 0)))

    logits, vals = pl.pallas_call(
        _fused_kernel,
        out_shape=(
            jax.ShapeDtypeStruct((B_pad, NUM_OUT), jnp.float32),
            jax.ShapeDtypeStruct((B_pad // LANE, LANE), jnp.float32),
        ),
        grid=(B_pad // tb,),
        in_specs=[
            pl.BlockSpec((tb, obs.shape[1]), lambda i: (i, 0)),
            pl.BlockSpec((tb, cobs.shape[1]), lambda i: (i, 0)),
            pl.BlockSpec(W1p.shape, lambda i: (0, 0)),
            pl.BlockSpec(W1v.shape, lambda i: (0, 0)),
            pl.BlockSpec((1, LANE), lambda i: (0, 0)),
            pl.BlockSpec((LANE, LANE), lambda i: (0, 0)),
            pl.BlockSpec((1, LANE), lambda i: (0, 0)),
            pl.BlockSpec((LANE, LANE), lambda i: (0, 0)),
            pl.BlockSpec((1, LANE), lambda i: (0, 0)),
        ],
        out_specs=(
            pl.BlockSpec((tb, NUM_OUT), lambda i: (i, 0)),
            pl.BlockSpec((tb // LANE, LANE), lambda i: (i, 0)),
        ),
        compiler_params=pltpu.CompilerParams(
            dimension_semantics=("parallel",),
        ),
    )(obs, cobs, W1p, W1v, B1, W2, B2, W3, B3)

    return logits[:B], vals.reshape(-1)[:B]


def kernel(obs, cc_obs, W1p, W1v, B1, W2, B2, W3, B3):
    return _impl(obs, cc_obs, W1p, W1v, B1, W2, B2, W3, B3)


# speedup vs baseline: 1.6440x; 1.2548x over previous
import functools

import jax
import jax.numpy as jnp
from jax.experimental import pallas as pl
from jax.experimental.pallas import tpu as pltpu

LANE = 128
NUM_OUT = 64
VAL_LANE = 64
MAX_BATCH_TILE = 2048


def _round_up(x, m):
    return ((x + m - 1) // m) * m


def _choose_tile(B):
    B128 = _round_up(max(B, 1), LANE)
    tb = min(MAX_BATCH_TILE, B128)
    if B128 // tb < 2 and B128 >= 2 * LANE:
        tb = _round_up((B128 + 1) // 2, LANE)
    B_pad = _round_up(B128, tb)
    return tb, B_pad


def _fused_kernel(obs_ref, cobs_ref, w1p_ref, w1v_ref, b1_ref,
                  w2_ref, b2_ref, w3_ref, b3_ref, logits_ref, vals_ref):
    tb = obs_ref.shape[0]
    obs = obs_ref[...].astype(jnp.bfloat16)
    cobs = cobs_ref[...].astype(jnp.bfloat16)
    w1p = w1p_ref[...].astype(jnp.bfloat16)
    w1v = w1v_ref[...].astype(jnp.bfloat16)
    w2 = w2_ref[...].astype(jnp.bfloat16)
    w3 = w3_ref[...].astype(jnp.bfloat16)
    pre = (jnp.dot(obs, w1p, preferred_element_type=jnp.float32)
           + jnp.dot(cobs, w1v, preferred_element_type=jnp.float32))
    h1 = jnp.tanh(pre + b1_ref[...]).astype(jnp.bfloat16)
    h2 = jnp.tanh(
        jnp.dot(h1, w2, preferred_element_type=jnp.float32)
        + b2_ref[...]).astype(jnp.bfloat16)
    logits_ref[...] = (
        jnp.dot(h2, w3[:, :NUM_OUT], preferred_element_type=jnp.float32)
        + b3_ref[0, :NUM_OUT])
    w3v_row = w3[:, VAL_LANE:VAL_LANE + 1].reshape(1, LANE)
    v_row = jax.lax.dot_general(
        w3v_row, h2, dimension_numbers=(((1,), (1,)), ((), ())),
        preferred_element_type=jnp.float32)
    vals_ref[...] = v_row.reshape(tb // LANE, LANE) + b3_ref[0, VAL_LANE]


@jax.jit
def _impl(obs, cc_obs, W1p, W1v, B1, W2, B2, W3, B3):
    obs = obs.astype(jnp.float32).reshape(obs.shape[0], -1)
    cobs = cc_obs.astype(jnp.float32).reshape(cc_obs.shape[0], -1)
    B = obs.shape[0]
    tb, B_pad = _choose_tile(B)
    if B_pad != B:
        obs = jnp.pad(obs, ((0, B_pad - B), (0, 0)))
        cobs = jnp.pad(cobs, ((0, B_pad - B), (0, 0)))

    logits, vals = pl.pallas_call(
        _fused_kernel,
        out_shape=(
            jax.ShapeDtypeStruct((B_pad, NUM_OUT), jnp.float32),
            jax.ShapeDtypeStruct((B_pad // LANE, LANE), jnp.float32),
        ),
        grid=(B_pad // tb,),
        in_specs=[
            pl.BlockSpec((tb, obs.shape[1]), lambda i: (i, 0)),
            pl.BlockSpec((tb, cobs.shape[1]), lambda i: (i, 0)),
            pl.BlockSpec(W1p.shape, lambda i: (0, 0)),
            pl.BlockSpec(W1v.shape, lambda i: (0, 0)),
            pl.BlockSpec((1, LANE), lambda i: (0, 0)),
            pl.BlockSpec((LANE, LANE), lambda i: (0, 0)),
            pl.BlockSpec((1, LANE), lambda i: (0, 0)),
            pl.BlockSpec((LANE, LANE), lambda i: (0, 0)),
            pl.BlockSpec((1, LANE), lambda i: (0, 0)),
        ],
        out_specs=(
            pl.BlockSpec((tb, NUM_OUT), lambda i: (i, 0)),
            pl.BlockSpec((tb // LANE, LANE), lambda i: (i, 0)),
        ),
        compiler_params=pltpu.CompilerParams(
            dimension_semantics=("parallel",),
        ),
    )(obs, cobs, W1p, W1v, B1, W2, B2, W3, B3)

    return logits[:B], vals.reshape(-1)[:B]


def kernel(obs, cc_obs, W1p, W1v, B1, W2, B2, W3, B3):
    return _impl(obs, cc_obs, W1p, W1v, B1, W2, B2, W3, B3)


# speedup vs baseline: 1.8836x; 1.1457x over previous
import functools

import jax
import jax.numpy as jnp
from jax.experimental import pallas as pl
from jax.experimental.pallas import tpu as pltpu

LANE = 128
NUM_OUT = 64
VAL_LANE = 64
MAX_BATCH_TILE = 4096


def _round_up(x, m):
    return ((x + m - 1) // m) * m


def _choose_tile(B):
    B128 = _round_up(max(B, 1), LANE)
    tb = min(MAX_BATCH_TILE, B128)
    if B128 // tb < 2 and B128 >= 2 * LANE:
        tb = _round_up((B128 + 1) // 2, LANE)
    B_pad = _round_up(B128, tb)
    return tb, B_pad


def _fused_kernel(obs_ref, cobs_ref, w1p_ref, w1v_ref, b1_ref,
                  w2_ref, b2_ref, w3_ref, b3_ref, logits_ref, vals_ref):
    tb = obs_ref.shape[0]
    obs = obs_ref[...].astype(jnp.bfloat16)
    cobs = cobs_ref[...].astype(jnp.bfloat16)
    w1p = w1p_ref[...].astype(jnp.bfloat16)
    w1v = w1v_ref[...].astype(jnp.bfloat16)
    w2 = w2_ref[...].astype(jnp.bfloat16)
    w3 = w3_ref[...].astype(jnp.bfloat16)
    pre = (jnp.dot(obs, w1p, preferred_element_type=jnp.float32)
           + jnp.dot(cobs, w1v, preferred_element_type=jnp.float32))
    h1 = jnp.tanh(pre + b1_ref[...]).astype(jnp.bfloat16)
    h2 = jnp.tanh(
        jnp.dot(h1, w2, preferred_element_type=jnp.float32)
        + b2_ref[...]).astype(jnp.bfloat16)
    logits_ref[...] = (
        jnp.dot(h2, w3[:, :NUM_OUT], preferred_element_type=jnp.float32)
        + b3_ref[0, :NUM_OUT])
    w3v_row = w3[:, VAL_LANE:VAL_LANE + 1].reshape(1, LANE)
    v_row = jax.lax.dot_general(
        w3v_row, h2, dimension_numbers=(((1,), (1,)), ((), ())),
        preferred_element_type=jnp.float32)
    vals_ref[...] = v_row.reshape(tb // LANE, LANE) + b3_ref[0, VAL_LANE]


@jax.jit
def _impl(obs, cc_obs, W1p, W1v, B1, W2, B2, W3, B3):
    obs = obs.astype(jnp.float32).reshape(obs.shape[0], -1)
    cobs = cc_obs.astype(jnp.float32).reshape(cc_obs.shape[0], -1)
    B = obs.shape[0]
    tb, B_pad = _choose_tile(B)
    if B_pad != B:
        obs = jnp.pad(obs, ((0, B_pad - B), (0, 0)))
        cobs = jnp.pad(cobs, ((0, B_pad - B), (0, 0)))

    logits, vals = pl.pallas_call(
        _fused_kernel,
        out_shape=(
            jax.ShapeDtypeStruct((B_pad, NUM_OUT), jnp.float32),
            jax.ShapeDtypeStruct((B_pad // LANE, LANE), jnp.float32),
        ),
        grid=(B_pad // tb,),
        in_specs=[
            pl.BlockSpec((tb, obs.shape[1]), lambda i: (i, 0)),
            pl.BlockSpec((tb, cobs.shape[1]), lambda i: (i, 0)),
            pl.BlockSpec(W1p.shape, lambda i: (0, 0)),
            pl.BlockSpec(W1v.shape, lambda i: (0, 0)),
            pl.BlockSpec((1, LANE), lambda i: (0, 0)),
            pl.BlockSpec((LANE, LANE), lambda i: (0, 0)),
            pl.BlockSpec((1, LANE), lambda i: (0, 0)),
            pl.BlockSpec((LANE, LANE), lambda i: (0, 0)),
            pl.BlockSpec((1, LANE), lambda i: (0, 0)),
        ],
        out_specs=(
            pl.BlockSpec((tb, NUM_OUT), lambda i: (i, 0)),
            pl.BlockSpec((tb // LANE, LANE), lambda i: (i, 0)),
        ),
        compiler_params=pltpu.CompilerParams(
            dimension_semantics=("parallel",),
        ),
    )(obs, cobs, W1p, W1v, B1, W2, B2, W3, B3)

    return logits[:B], vals.reshape(-1)[:B]


def kernel(obs, cc_obs, W1p, W1v, B1, W2, B2, W3, B3):
    return _impl(obs, cc_obs, W1p, W1v, B1, W2, B2, W3, B3)


# speedup vs baseline: 1.9343x; 1.0269x over previous
import functools

import jax
import jax.numpy as jnp
from jax.experimental import pallas as pl
from jax.experimental.pallas import tpu as pltpu

LANE = 128
NUM_OUT = 64
VAL_LANE = 64
MAX_BATCH_TILE = 8192


def _round_up(x, m):
    return ((x + m - 1) // m) * m


def _choose_tile(B):
    B128 = _round_up(max(B, 1), LANE)
    tb = min(MAX_BATCH_TILE, B128)
    if B128 // tb < 2 and B128 >= 2 * LANE:
        tb = _round_up((B128 + 1) // 2, LANE)
    B_pad = _round_up(B128, tb)
    return tb, B_pad


def _fused_kernel(obs_ref, cobs_ref, w1p_ref, w1v_ref, b1_ref,
                  w2_ref, b2_ref, w3_ref, b3_ref, logits_ref, vals_ref):
    tb = obs_ref.shape[0]
    obs = obs_ref[...].astype(jnp.bfloat16)
    cobs = cobs_ref[...].astype(jnp.bfloat16)
    w1p = w1p_ref[...].astype(jnp.bfloat16)
    w1v = w1v_ref[...].astype(jnp.bfloat16)
    w2 = w2_ref[...].astype(jnp.bfloat16)
    w3 = w3_ref[...].astype(jnp.bfloat16)
    pre = (jnp.dot(obs, w1p, preferred_element_type=jnp.float32)
           + jnp.dot(cobs, w1v, preferred_element_type=jnp.float32))
    h1 = jnp.tanh(pre + b1_ref[...]).astype(jnp.bfloat16)
    h2 = jnp.tanh(
        jnp.dot(h1, w2, preferred_element_type=jnp.float32)
        + b2_ref[...]).astype(jnp.bfloat16)
    logits_ref[...] = (
        jnp.dot(h2, w3[:, :NUM_OUT], preferred_element_type=jnp.float32)
        + b3_ref[0, :NUM_OUT])
    w3v_row = w3[:, VAL_LANE:VAL_LANE + 1].reshape(1, LANE)
    v_row = jax.lax.dot_general(
        w3v_row, h2, dimension_numbers=(((1,), (1,)), ((), ())),
        preferred_element_type=jnp.float32)
    vals_ref[...] = v_row.reshape(tb // LANE, LANE) + b3_ref[0, VAL_LANE]


@jax.jit
def _impl(obs, cc_obs, W1p, W1v, B1, W2, B2, W3, B3):
    obs = obs.astype(jnp.float32).reshape(obs.shape[0], -1)
    cobs = cc_obs.astype(jnp.float32).reshape(cc_obs.shape[0], -1)
    B = obs.shape[0]
    tb, B_pad = _choose_tile(B)
    if B_pad != B:
        obs = jnp.pad(obs, ((0, B_pad - B), (0, 0)))
        cobs = jnp.pad(cobs, ((0, B_pad - B), (0, 0)))

    logits, vals = pl.pallas_call(
        _fused_kernel,
        out_shape=(
            jax.ShapeDtypeStruct((B_pad, NUM_OUT), jnp.float32),
            jax.ShapeDtypeStruct((B_pad // LANE, LANE), jnp.float32),
        ),
        grid=(B_pad // tb,),
        in_specs=[
            pl.BlockSpec((tb, obs.shape[1]), lambda i: (i, 0)),
            pl.BlockSpec((tb, cobs.shape[1]), lambda i: (i, 0)),
            pl.BlockSpec(W1p.shape, lambda i: (0, 0)),
            pl.BlockSpec(W1v.shape, lambda i: (0, 0)),
            pl.BlockSpec((1, LANE), lambda i: (0, 0)),
            pl.BlockSpec((LANE, LANE), lambda i: (0, 0)),
            pl.BlockSpec((1, LANE), lambda i: (0, 0)),
            pl.BlockSpec((LANE, LANE), lambda i: (0, 0)),
            pl.BlockSpec((1, LANE), lambda i: (0, 0)),
        ],
        out_specs=(
            pl.BlockSpec((tb, NUM_OUT), lambda i: (i, 0)),
            pl.BlockSpec((tb // LANE, LANE), lambda i: (i, 0)),
        ),
        compiler_params=pltpu.CompilerParams(
            dimension_semantics=("parallel",),
        ),
    )(obs, cobs, W1p, W1v, B1, W2, B2, W3, B3)

    return logits[:B], vals.reshape(-1)[:B]


def kernel(obs, cc_obs, W1p, W1v, B1, W2, B2, W3, B3):
    return _impl(obs, cc_obs, W1p, W1v, B1, W2, B2, W3, B3)


# speedup vs baseline: 1.9367x; 1.0012x over previous
import functools

import jax
import jax.numpy as jnp
from jax.experimental import pallas as pl
from jax.experimental.pallas import tpu as pltpu

LANE = 128
NUM_OUT = 64
VAL_LANE = 64
MAX_BATCH_TILE = 8192


def _round_up(x, m):
    return ((x + m - 1) // m) * m


def _choose_tile(B):
    B128 = _round_up(max(B, 1), LANE)
    tb = min(MAX_BATCH_TILE, B128)
    if B128 // tb < 2 and B128 >= 2 * LANE:
        tb = _round_up((B128 + 1) // 2, LANE)
    B_pad = _round_up(B128, tb)
    return tb, B_pad


def _fused_kernel(obs_ref, cobs_ref, w1p_ref, w1v_ref, b1_ref,
                  w2_ref, b2_ref, w3_ref, b3_ref, logits_ref, vals_ref):
    tb = obs_ref.shape[0]
    obs = obs_ref[...].astype(jnp.bfloat16)
    cobs = cobs_ref[...].astype(jnp.bfloat16)
    w1p = w1p_ref[...].astype(jnp.bfloat16)
    w1v = w1v_ref[...].astype(jnp.bfloat16)
    w2 = w2_ref[...].astype(jnp.bfloat16)
    w3 = w3_ref[...].astype(jnp.bfloat16)
    pre = (jnp.dot(obs, w1p, preferred_element_type=jnp.float32)
           + jnp.dot(cobs, w1v, preferred_element_type=jnp.float32))
    h1 = jnp.tanh(pre + b1_ref[...]).astype(jnp.bfloat16)
    h2 = jnp.tanh(
        jnp.dot(h1, w2, preferred_element_type=jnp.float32)
        + b2_ref[...]).astype(jnp.bfloat16)
    logits_ref[...] = (
        jnp.dot(h2, w3[:, :NUM_OUT], preferred_element_type=jnp.float32)
        + b3_ref[0, :NUM_OUT])
    w3v_row = w3[:, VAL_LANE:VAL_LANE + 1].reshape(1, LANE)
    v_row = jax.lax.dot_general(
        w3v_row, h2, dimension_numbers=(((1,), (1,)), ((), ())),
        preferred_element_type=jnp.float32)
    vals_ref[...] = v_row.reshape(tb // LANE, LANE) + b3_ref[0, VAL_LANE]


@jax.jit
def _impl(obs, cc_obs, W1p, W1v, B1, W2, B2, W3, B3):
    obs = obs.astype(jnp.float32).reshape(obs.shape[0], -1)
    cobs = cc_obs.astype(jnp.float32).reshape(cc_obs.shape[0], -1)
    B = obs.shape[0]
    tb, B_pad = _choose_tile(B)
    if B_pad != B:
        obs = jnp.pad(obs, ((0, B_pad - B), (0, 0)))
        cobs = jnp.pad(cobs, ((0, B_pad - B), (0, 0)))

    logits, vals = pl.pallas_call(
        _fused_kernel,
        out_shape=(
            jax.ShapeDtypeStruct((B_pad, NUM_OUT), jnp.float32),
            jax.ShapeDtypeStruct((B_pad // LANE, LANE), jnp.float32),
        ),
        grid=(B_pad // tb,),
        in_specs=[
            pl.BlockSpec((tb, obs.shape[1]), lambda i: (i, 0)),
            pl.BlockSpec((tb, cobs.shape[1]), lambda i: (i, 0)),
            pl.BlockSpec(W1p.shape, lambda i: (0, 0)),
            pl.BlockSpec(W1v.shape, lambda i: (0, 0)),
            pl.BlockSpec((1, LANE), lambda i: (0, 0)),
            pl.BlockSpec((LANE, LANE), lambda i: (0, 0)),
            pl.BlockSpec((1, LANE), lambda i: (0, 0)),
            pl.BlockSpec((LANE, LANE), lambda i: (0, 0)),
            pl.BlockSpec((1, LANE), lambda i: (0, 0)),
        ],
        out_specs=(
            pl.BlockSpec((tb, NUM_OUT), lambda i: (i, 0)),
            pl.BlockSpec((tb // LANE, LANE), lambda i: (i, 0)),
        ),
        compiler_params=pltpu.CompilerParams(
            dimension_semantics=("parallel",),
        ),
    )(obs, cobs, W1p, W1v, B1, W2, B2, W3, B3)

    values = vals.reshape(-1)
    if B_pad != B:
        return logits[:B], values[:B]
    return logits, values


def kernel(obs, cc_obs, W1p, W1v, B1, W2, B2, W3, B3):
    return _impl(obs, cc_obs, W1p, W1v, B1, W2, B2, W3, B3)


# speedup vs baseline: 3.3598x; 1.7348x over previous
import functools

import jax
import jax.numpy as jnp
from jax.experimental import pallas as pl
from jax.experimental.pallas import tpu as pltpu

LANE = 128
NUM_OUT = 64
VAL_LANE = 64
MAX_BATCH_TILE = 8192


def _round_up(x, m):
    return ((x + m - 1) // m) * m


def _choose_tile(B):
    B128 = _round_up(max(B, 1), LANE)
    tb = min(MAX_BATCH_TILE, B128)
    if B128 // tb < 2 and B128 >= 2 * LANE:
        tb = _round_up((B128 + 1) // 2, LANE)
    B_pad = _round_up(B128, tb)
    return tb, B_pad


def _fused_kernel(obs_ref, cobs_ref, w1p_ref, w1v_ref, b1_ref,
                  w2_ref, b2_ref, w3_ref, b3_ref, logits_ref, vals_ref):
    tb = obs_ref.shape[0]
    obs = obs_ref[...].astype(jnp.bfloat16)
    cobs = cobs_ref[...].astype(jnp.bfloat16)
    w1p = w1p_ref[...].astype(jnp.bfloat16)
    w1v = w1v_ref[...].astype(jnp.bfloat16)
    w2 = w2_ref[...].astype(jnp.bfloat16)
    w3 = w3_ref[...].astype(jnp.bfloat16)
    pre = (jnp.dot(obs, w1p, preferred_element_type=jnp.float32)
           + jnp.dot(cobs, w1v, preferred_element_type=jnp.float32))
    h1 = jnp.tanh(pre + b1_ref[...]).astype(jnp.bfloat16)
    h2 = jnp.tanh(
        jnp.dot(h1, w2, preferred_element_type=jnp.float32)
        + b2_ref[...]).astype(jnp.bfloat16)
    out_t = jax.lax.dot_general(
        w3, h2, dimension_numbers=(((0,), (1,)), ((), ())),
        preferred_element_type=jnp.float32)
    b3_t = b3_ref[...].reshape(LANE, 1)
    logits_ref[...] = out_t[:NUM_OUT, :] + b3_t[:NUM_OUT]
    v_row = out_t[VAL_LANE:VAL_LANE + 1, :] + b3_ref[0, VAL_LANE]
    vals_ref[...] = v_row.reshape(tb // LANE, LANE)


@jax.jit
def _impl(obs, cc_obs, W1p, W1v, B1, W2, B2, W3, B3):
    obs = obs.astype(jnp.float32).reshape(obs.shape[0], -1)
    cobs = cc_obs.astype(jnp.float32).reshape(cc_obs.shape[0], -1)
    B = obs.shape[0]
    tb, B_pad = _choose_tile(B)
    if B_pad != B:
        obs = jnp.pad(obs, ((0, B_pad - B), (0, 0)))
        cobs = jnp.pad(cobs, ((0, B_pad - B), (0, 0)))

    logits, vals = pl.pallas_call(
        _fused_kernel,
        out_shape=(
            jax.ShapeDtypeStruct((NUM_OUT, B_pad), jnp.float32),
            jax.ShapeDtypeStruct((B_pad // LANE, LANE), jnp.float32),
        ),
        grid=(B_pad // tb,),
        in_specs=[
            pl.BlockSpec((tb, obs.shape[1]), lambda i: (i, 0)),
            pl.BlockSpec((tb, cobs.shape[1]), lambda i: (i, 0)),
            pl.BlockSpec(W1p.shape, lambda i: (0, 0)),
            pl.BlockSpec(W1v.shape, lambda i: (0, 0)),
            pl.BlockSpec((1, LANE), lambda i: (0, 0)),
            pl.BlockSpec((LANE, LANE), lambda i: (0, 0)),
            pl.BlockSpec((1, LANE), lambda i: (0, 0)),
            pl.BlockSpec((LANE, LANE), lambda i: (0, 0)),
            pl.BlockSpec((1, LANE), lambda i: (0, 0)),
        ],
        out_specs=(
            pl.BlockSpec((NUM_OUT, tb), lambda i: (0, i)),
            pl.BlockSpec((tb // LANE, LANE), lambda i: (i, 0)),
        ),
        compiler_params=pltpu.CompilerParams(
            dimension_semantics=("parallel",),
        ),
    )(obs, cobs, W1p, W1v, B1, W2, B2, W3, B3)

    logits = logits.T
    values = vals.reshape(-1)
    if B_pad != B:
        return logits[:B], values[:B]
    return logits, values


def kernel(obs, cc_obs, W1p, W1v, B1, W2, B2, W3, B3):
    return _impl(obs, cc_obs, W1p, W1v, B1, W2, B2, W3, B3)
